# Initial kernel scaffold; baseline (speedup 1.0000x reference)
import jax
import jax.numpy as jnp
from jax.experimental import pallas as pl


def kernel(x, edge_index, W1, b1, W2, b2):
    raise NotImplementedError("write your pallas kernel here")


# speedup vs baseline: 10.5615x; 10.5615x over previous
import functools

import jax
import jax.numpy as jnp
from jax import lax
from jax.experimental import pallas as pl
from jax.experimental.pallas import tpu as pltpu
from jax.experimental.pallas import tpu_sc as plsc

N = 10000
E = 320000
IN_CH = 128
HID = 256
OUT_CH = 128

NC = 2
NS = 16
CH = 80
NPAD = 10240
ZR = 128
RPT = NPAD // NS

_MESH = dict(core_axis_name="c", subcore_axis_name="s", num_cores=NC,
             num_subcores=NS)


def _fill(ref, nrows, ncols, value):
    vec = jnp.full((16,), value, jnp.float32)

    def body(i, _):
        for t in range(ncols // 16):
            ref[i, pl.ds(t * 16, 16)] = vec
        return 0

    lax.fori_loop(0, nrows, body, 0)


@functools.cache
def _make_deg_kernel():
    cps = E // (NC * NS)
    nch = cps // CH

    @functools.partial(
        pl.kernel,
        out_type=jax.ShapeDtypeStruct((NC, NPAD, 16), jnp.float32),
        mesh=plsc.VectorSubcoreMesh(**_MESH),
        scratch_types=[
            pltpu.VMEM((CH,), jnp.int32),
            pltpu.VMEM((CH, 16), jnp.float32),
            pltpu.VMEM((ZR, 16), jnp.float32),
            pltpu.VMEM_SHARED((NPAD, 16), jnp.float32),
        ],
    )
    def deg(dst_hbm, out_hbm, didx, ones, zbuf, acc):
        c = lax.axis_index("c")
        s = lax.axis_index("s")
        _fill(ones, CH, 16, 1.0)
        _fill(zbuf, ZR, 16, 0.0)
        for j in range(RPT // ZR):
            pltpu.sync_copy(zbuf, acc.at[pl.ds(s * RPT + j * ZR, ZR)])
        plsc.subcore_barrier()

        ebase = (c * NS + s) * cps

        def chunk(j, _):
            pltpu.sync_copy(dst_hbm.at[pl.ds(ebase + j * CH, CH)], didx)
            pltpu.sync_copy(ones, acc.at[didx], add=True)
            return 0

        lax.fori_loop(0, nch, chunk, 0)
        plsc.subcore_barrier()
        for j in range(RPT // ZR):
            r0 = s * RPT + j * ZR
            pltpu.sync_copy(acc.at[pl.ds(r0, ZR)], zbuf)
            pltpu.sync_copy(zbuf, out_hbm.at[c, pl.ds(r0, ZR)])

    return deg


@functools.cache
def _make_agg_kernel(w, split_channels):
    cps = E // NS if split_channels else E // (NC * NS)
    nch = cps // CH

    @functools.partial(
        pl.kernel,
        out_type=jax.ShapeDtypeStruct((NC, NPAD, w), jnp.float32),
        mesh=plsc.VectorSubcoreMesh(**_MESH),
        scratch_types=[
            pltpu.VMEM((CH,), jnp.int32),
            pltpu.VMEM((CH,), jnp.int32),
            pltpu.VMEM((CH, w), jnp.float32),
            pltpu.VMEM((ZR, w), jnp.float32),
            pltpu.VMEM_SHARED((NPAD, w), jnp.float32),
            pltpu.SemaphoreType.DMA,
        ],
    )
    def agg(ycat_hbm, src_hbm, dst_hbm, out_hbm, sidx, didx, rows, zbuf, acc,
            sem):
        c = lax.axis_index("c")
        s = lax.axis_index("s")
        _fill(zbuf, ZR, w, 0.0)
        for j in range(RPT // ZR):
            pltpu.sync_copy(zbuf, acc.at[pl.ds(s * RPT + j * ZR, ZR)])
        plsc.subcore_barrier()

        if split_channels:
            coff = c * N
            ebase = s * cps
        else:
            coff = 0
            ebase = (c * NS + s) * cps

        def chunk(j, _):
            e0 = ebase + j * CH
            pltpu.sync_copy(src_hbm.at[pl.ds(e0, CH)], sidx)
            pltpu.sync_copy(dst_hbm.at[pl.ds(e0, CH)], didx)
            if split_channels:
                for t in range(CH // 16):
                    sl = pl.ds(t * 16, 16)
                    sidx[sl] = sidx[sl] + coff
            pltpu.async_copy(ycat_hbm.at[sidx], rows, sem).wait()
            pltpu.sync_copy(rows, acc.at[didx], add=True)
            return 0

        lax.fori_loop(0, nch, chunk, 0)
        plsc.subcore_barrier()
        for j in range(RPT // ZR):
            r0 = s * RPT + j * ZR
            pltpu.sync_copy(acc.at[pl.ds(r0, ZR)], zbuf)
            pltpu.sync_copy(zbuf, out_hbm.at[c, pl.ds(r0, ZR)])

    return agg


_BN = 1000
_GRID = (N // _BN,)


def _tc_prep_body(x_ref, w_ref, cnt_ref, ycat_ref, d_ref):
    deg = cnt_ref[0, :, 0:1] + cnt_ref[1, :, 0:1] + 1.0
    dv = lax.rsqrt(deg)
    mm = jnp.dot(x_ref[...], w_ref[...], preferred_element_type=jnp.float32)
    y = mm * dv
    ycat_ref[0] = y[:, :IN_CH]
    ycat_ref[1] = y[:, IN_CH:]
    d_ref[...] = jnp.broadcast_to(dv, (_BN, IN_CH))


def _tc_prep(x, w1, cnt):
    return pl.pallas_call(
        _tc_prep_body,
        grid=_GRID,
        in_specs=[
            pl.BlockSpec((_BN, IN_CH), lambda i: (i, 0)),
            pl.BlockSpec((IN_CH, HID), lambda i: (0, 0)),
            pl.BlockSpec((NC, _BN, 16), lambda i: (0, i, 0)),
        ],
        out_specs=[
            pl.BlockSpec((NC, _BN, IN_CH), lambda i: (0, i, 0)),
            pl.BlockSpec((_BN, IN_CH), lambda i: (i, 0)),
        ],
        out_shape=[
            jax.ShapeDtypeStruct((NC, N, IN_CH), jnp.float32),
            jax.ShapeDtypeStruct((N, IN_CH), jnp.float32),
        ],
    )(x, w1, cnt)


def _tc_mid_body(agg_ref, y_ref, d_ref, b_ref, w_ref, out_ref):
    d = d_ref[...]
    h0 = jnp.maximum(d * (agg_ref[0] + y_ref[0]) + b_ref[0], 0.0)
    h1 = jnp.maximum(d * (agg_ref[1] + y_ref[1]) + b_ref[1], 0.0)
    h = jnp.concatenate([h0, h1], axis=1)
    mm = jnp.dot(h, w_ref[...], preferred_element_type=jnp.float32)
    out_ref[...] = mm * d


def _tc_mid(agg1, ycat1, d, b1r, w2):
    return pl.pallas_call(
        _tc_mid_body,
        grid=_GRID,
        in_specs=[
            pl.BlockSpec((NC, _BN, IN_CH), lambda i: (0, i, 0)),
            pl.BlockSpec((NC, _BN, IN_CH), lambda i: (0, i, 0)),
            pl.BlockSpec((_BN, IN_CH), lambda i: (i, 0)),
            pl.BlockSpec((NC, 1, IN_CH), lambda i: (0, 0, 0)),
            pl.BlockSpec((HID, OUT_CH), lambda i: (0, 0)),
        ],
        out_specs=pl.BlockSpec((_BN, OUT_CH), lambda i: (i, 0)),
        out_shape=jax.ShapeDtypeStruct((N, OUT_CH), jnp.float32),
    )(agg1, ycat1, d, b1r, w2)


def _tc_out_body(agg_ref, y_ref, d_ref, b_ref, out_ref):
    d = d_ref[...]
    s = agg_ref[0] + agg_ref[1] + y_ref[...]
    out_ref[...] = jnp.maximum(d * s + b_ref[...], 0.0)


def _tc_out(agg2, y2, d, b2r):
    return pl.pallas_call(
        _tc_out_body,
        grid=_GRID,
        in_specs=[
            pl.BlockSpec((NC, _BN, OUT_CH), lambda i: (0, i, 0)),
            pl.BlockSpec((_BN, OUT_CH), lambda i: (i, 0)),
            pl.BlockSpec((_BN, IN_CH), lambda i: (i, 0)),
            pl.BlockSpec((1, OUT_CH), lambda i: (0, 0)),
        ],
        out_specs=pl.BlockSpec((_BN, OUT_CH), lambda i: (i, 0)),
        out_shape=jax.ShapeDtypeStruct((N, OUT_CH), jnp.float32),
    )(agg2, y2, d, b2r)


@jax.jit
def kernel(x, edge_index, W1, b1, W2, b2):
    src = edge_index[0]
    dst = edge_index[1]
    cnt = _make_deg_kernel()(dst)
    ycat1, d = _tc_prep(x, W1, cnt)
    agg1 = _make_agg_kernel(IN_CH, True)(
        ycat1.reshape(NC * N, IN_CH), src, dst)
    y2 = _tc_mid(agg1, ycat1, d, b1.reshape(NC, 1, IN_CH), W2)
    agg2 = _make_agg_kernel(OUT_CH, False)(y2, src, dst)
    return _tc_out(agg2, y2, d, b2.reshape(1, OUT_CH))
